# baseline (speedup 1.0000x reference)
import functools

import jax
import jax.numpy as jnp
from jax import lax
from jax.experimental import pallas as pl
from jax.experimental.pallas import tpu as pltpu
from jax.experimental.pallas import tpu_sc as plsc

HID = 128
EMB = 64
TBINS = 40
EBINS = 80
NAL = 4
NGCN = 4

NSC = 2
NTILE = 16
NW = NSC * NTILE

BATCH = 80
RPC = 8
CHUNK = BATCH * RPC
TBLK = 1000
ZROWS = 2500

_f32 = jnp.float32


def _ln(h, g, b):
    mu = jnp.mean(h, axis=-1, keepdims=True)
    var = jnp.mean((h - mu) ** 2, axis=-1, keepdims=True)
    return (h - mu) / jnp.sqrt(var + 1e-5) * g + b


def _rbf_body(v_ref, w1, b1, g1, e1, w2, b2, g2, e2, o_ref, *, vmin, vmax, bins):
    v = v_ref[...]
    step = (vmax - vmin) / (bins - 1)
    centers = vmin + lax.broadcasted_iota(jnp.int32, (1, bins), 1).astype(_f32) * step
    gamma = 0.5 / ((vmax - vmin) / bins) ** 2
    f = jnp.exp(-gamma * (v - centers) ** 2)
    h = jnp.dot(f, w1[...], preferred_element_type=_f32) + b1[...]
    h = jax.nn.silu(_ln(h, g1[...], e1[...]))
    h = jnp.dot(h, w2[...], preferred_element_type=_f32) + b2[...]
    o_ref[...] = jax.nn.silu(_ln(h, g2[...], e2[...]))


@functools.lru_cache(maxsize=None)
def _rbf_embed_call(n, bins, vmin, vmax):
    full = lambda shape: pl.BlockSpec(shape, lambda i: (0, 0))
    return pl.pallas_call(
        functools.partial(_rbf_body, vmin=vmin, vmax=vmax, bins=bins),
        grid=(n // TBLK,),
        in_specs=[
            pl.BlockSpec((TBLK, 1), lambda i: (i, 0)),
            full((bins, EMB)), full((1, EMB)), full((1, EMB)), full((1, EMB)),
            full((EMB, HID)), full((1, HID)), full((1, HID)), full((1, HID)),
        ],
        out_specs=pl.BlockSpec((TBLK, HID), lambda i: (i, 0)),
        out_shape=jax.ShapeDtypeStruct((n, HID), _f32),
    )


def _rbf_embed(vals, p1, p2, vmin, vmax, bins):
    n = vals.shape[0]
    return _rbf_embed_call(n, bins, vmin, vmax)(
        vals.reshape(n, 1),
        p1["lin"]["W"], p1["lin"]["b"][None], p1["ln"]["g"][None], p1["ln"]["b"][None],
        p2["lin"]["W"], p2["lin"]["b"][None], p2["ln"]["g"][None], p2["ln"]["b"][None],
    )


def _atom_body(x_ref, w, b, g, e, o_ref):
    h = jnp.dot(x_ref[...], w[...], preferred_element_type=_f32) + b[...]
    o_ref[...] = jax.nn.silu(_ln(h, g[...], e[...]))


@functools.lru_cache(maxsize=None)
def _atom_embed_call(n, din):
    full = lambda shape: pl.BlockSpec(shape, lambda i: (0, 0))
    return pl.pallas_call(
        _atom_body,
        grid=(n // TBLK,),
        in_specs=[
            pl.BlockSpec((TBLK, din), lambda i: (i, 0)),
            full((din, HID)), full((1, HID)), full((1, HID)), full((1, HID)),
        ],
        out_specs=pl.BlockSpec((TBLK, HID), lambda i: (i, 0)),
        out_shape=jax.ShapeDtypeStruct((n, HID), _f32),
    )


def _atom_embed(x, p):
    n, din = x.shape
    return _atom_embed_call(n, din)(
        x, p["lin"]["W"], p["lin"]["b"][None], p["ln"]["g"][None], p["ln"]["b"][None])


def _pass1_body(nf_ref, wcat, bcat, oa, od, ob, os_):
    h = jnp.dot(nf_ref[...], wcat[...], preferred_element_type=_f32) + bcat[...]
    oa[...] = h[:, 0 * HID:1 * HID]
    od[...] = h[:, 1 * HID:2 * HID]
    ob[...] = h[:, 2 * HID:3 * HID]
    os_[...] = h[:, 3 * HID:4 * HID]


@functools.lru_cache(maxsize=None)
def _pass1_call(n):
    full = lambda shape: pl.BlockSpec(shape, lambda i: (0, 0))
    blk = pl.BlockSpec((TBLK, HID), lambda i: (i, 0))
    return pl.pallas_call(
        _pass1_body,
        grid=(n // TBLK,),
        in_specs=[blk, full((HID, 4 * HID)), full((1, 4 * HID))],
        out_specs=[blk, blk, blk, blk],
        out_shape=[jax.ShapeDtypeStruct((n, HID), _f32)] * 4,
    )


def _pass2_body(as_ref, dd_ref, bs_ref, ef_ref, weg, beg, ge, be, us_ref, eo_ref):
    ef = ef_ref[...]
    eg = jnp.dot(ef, weg[...], preferred_element_type=_f32) + beg[...]
    sig = jax.nn.sigmoid(as_ref[...] + dd_ref[...] + eg)
    u = bs_ref[...] * sig
    us_ref[...] = jnp.concatenate([u, sig], axis=1)
    eo_ref[...] = _ln(sig * ef, ge[...], be[...])


@functools.lru_cache(maxsize=None)
def _pass2_call(E):
    full = lambda shape: pl.BlockSpec(shape, lambda i: (0, 0))
    blk = pl.BlockSpec((TBLK, HID), lambda i: (i, 0))
    return pl.pallas_call(
        _pass2_body,
        grid=(E // TBLK,),
        in_specs=[blk, blk, blk, blk,
                  full((HID, HID)), full((1, HID)), full((1, HID)), full((1, HID))],
        out_specs=[pl.BlockSpec((TBLK, 2 * HID), lambda i: (i, 0)), blk],
        out_shape=[jax.ShapeDtypeStruct((E, 2 * HID), _f32),
                   jax.ShapeDtypeStruct((E, HID), _f32)],
    )


def _pass3_body(s_ref, sums_ref, gn, bn, o_ref):
    sums = sums_ref[...]
    h = sums[:, :HID] / (sums[:, HID:] + 1e-8)
    o_ref[...] = _ln(s_ref[...] + h, gn[...], bn[...])


@functools.lru_cache(maxsize=None)
def _pass3_call(n):
    full = lambda shape: pl.BlockSpec(shape, lambda i: (0, 0))
    return pl.pallas_call(
        _pass3_body,
        grid=(n // TBLK,),
        in_specs=[pl.BlockSpec((TBLK, HID), lambda i: (i, 0)),
                  pl.BlockSpec((TBLK, 2 * HID), lambda i: (i, 0)),
                  full((1, HID)), full((1, HID))],
        out_specs=pl.BlockSpec((TBLK, HID), lambda i: (i, 0)),
        out_shape=jax.ShapeDtypeStruct((n, HID), _f32),
    )


def _mean_body(x_ref, o_ref, *, nblocks, n):
    i = pl.program_id(0)

    @pl.when(i == 0)
    def _():
        o_ref[...] = jnp.zeros_like(o_ref)

    o_ref[...] += jnp.sum(x_ref[...], axis=0, keepdims=True)

    @pl.when(i == nblocks - 1)
    def _():
        o_ref[...] *= 1.0 / n


@functools.lru_cache(maxsize=None)
def _mean_call(n):
    return pl.pallas_call(
        functools.partial(_mean_body, nblocks=n // TBLK, n=n),
        grid=(n // TBLK,),
        in_specs=[pl.BlockSpec((TBLK, HID), lambda i: (i, 0))],
        out_specs=pl.BlockSpec((1, HID), lambda i: (0, 0)),
        out_shape=jax.ShapeDtypeStruct((1, HID), _f32),
    )


def _sc_gather_body(A, D, B, src2, dst2, oA, oD, oB, sidx, didx, stag, sem,
                    *, nchunks, iters):
    cid = lax.axis_index("c")
    tid = lax.axis_index("s")
    wid = tid * NSC + cid

    def chunk_body(j, carry):
        k = j * NW + wid

        @pl.when(k < nchunks)
        def _():
            base = k * CHUNK
            row0 = k * RPC
            pltpu.sync_copy(src2.at[pl.ds(row0, RPC), :], sidx)
            pltpu.sync_copy(dst2.at[pl.ds(row0, RPC), :], didx)
            for tbl, idx, out in ((A, sidx, oA), (D, didx, oD), (B, sidx, oB)):
                descs = [
                    pltpu.async_copy(tbl.at[idx.at[r]],
                                     stag.at[pl.ds(r * BATCH, BATCH), :], sem)
                    for r in range(RPC)
                ]
                for d in descs:
                    d.wait()
                pltpu.sync_copy(stag, out.at[pl.ds(base, CHUNK), :])

        return carry

    lax.fori_loop(0, iters, chunk_body, 0)


@functools.lru_cache(maxsize=None)
def _sc_gather_call(n, E):
    nchunks = E // CHUNK
    iters = -(-nchunks // NW)
    mesh = plsc.VectorSubcoreMesh(core_axis_name="c", subcore_axis_name="s")
    return pl.kernel(
        functools.partial(_sc_gather_body, nchunks=nchunks, iters=iters),
        out_type=[jax.ShapeDtypeStruct((E, HID), _f32)] * 3,
        mesh=mesh,
        scratch_types=[
            pltpu.VMEM((RPC, BATCH), jnp.int32),
            pltpu.VMEM((RPC, BATCH), jnp.int32),
            pltpu.VMEM((CHUNK, HID), _f32),
            pltpu.SemaphoreType.DMA,
        ],
    )


def _sc_scatter_body(us, dst2, zrs, out, idxr, idxt, val, acc, sem,
                     *, nchunks, iters, half, segsplit, stripe):
    cid = lax.axis_index("c")
    tid = lax.axis_index("s")
    lo = cid * half if segsplit else 0
    ngroups = 16 if segsplit else 8

    for gg in range(ngroups):
        if segsplit:
            col0 = gg * 16
        else:
            col0 = (gg * NSC + cid) * 16
        nz = -(-stripe // ZROWS)
        for tz in range(nz):
            zr = min(ZROWS, stripe - tz * ZROWS)
            pltpu.sync_copy(zrs.at[pl.ds(0, zr), :],
                            acc.at[pl.ds(tid * stripe + tz * ZROWS, zr), :])
        plsc.subcore_barrier()

        def chunk_body(j, carry):
            k = j * NTILE + tid

            @pl.when(k < nchunks)
            def _():
                row0 = k * RPC
                base = k * CHUNK
                pltpu.sync_copy(dst2.at[pl.ds(row0, RPC), :], idxr)
                pltpu.sync_copy(us.at[pl.ds(base, CHUNK), pl.ds(col0, 16)], val)
                if segsplit:
                    for r in range(RPC):
                        def route(c, cc):
                            v = idxr[r, pl.ds(c * 16, 16)]
                            vl = v - lo
                            ok = (vl >= 0) & (vl < half)
                            idxt[r, pl.ds(c * 16, 16)] = jnp.where(ok, vl, half)
                            return cc
                        lax.fori_loop(0, BATCH // 16, route, 0)
                    idxs = idxt
                else:
                    idxs = idxr
                for r in range(RPC):
                    pltpu.sync_copy(val.at[pl.ds(r * BATCH, BATCH), :],
                                    acc.at[idxs.at[r]], add=True)

            return carry

        lax.fori_loop(0, iters, chunk_body, 0)
        plsc.subcore_barrier()
        pltpu.sync_copy(acc.at[pl.ds(tid * stripe, stripe), :],
                        out.at[pl.ds(lo + tid * stripe, stripe), pl.ds(col0, 16)])
        plsc.subcore_barrier()


@functools.lru_cache(maxsize=None)
def _sc_scatter_call(nseg, E, segsplit):
    nchunks = E // CHUNK
    iters = -(-nchunks // NTILE)
    half = nseg // NSC if segsplit else nseg
    accrows = half + 8 if segsplit else nseg
    stripe = half // NTILE
    mesh = plsc.VectorSubcoreMesh(core_axis_name="c", subcore_axis_name="s")
    return pl.kernel(
        functools.partial(_sc_scatter_body, nchunks=nchunks, iters=iters,
                          half=half, segsplit=segsplit, stripe=stripe),
        out_type=jax.ShapeDtypeStruct((nseg, 2 * HID), _f32),
        mesh=mesh,
        scratch_types=[
            pltpu.VMEM((RPC, BATCH), jnp.int32),
            pltpu.VMEM((RPC, BATCH), jnp.int32),
            pltpu.VMEM((CHUNK, 16), _f32),
            pltpu.VMEM_SHARED((accrows, 16), _f32),
            pltpu.SemaphoreType.DMA,
        ],
        compiler_params=pltpu.CompilerParams(use_tc_tiling_on_sc=False),
    )


def _egg(p, src2, dst2, nf, ef, nseg, E, zrs, segsplit):
    wcat = jnp.concatenate([p["src_gate"]["W"], p["dst_gate"]["W"],
                            p["dst_update"]["W"], p["src_update"]["W"]], axis=1)
    bcat = jnp.concatenate([p["src_gate"]["b"], p["dst_gate"]["b"],
                            p["dst_update"]["b"], p["src_update"]["b"]])[None]
    A, D, B, S = _pass1_call(nseg)(nf, wcat, bcat)
    As, Dd, Bs = _sc_gather_call(nseg, E)(A, D, B, src2, dst2)
    US, e_out = _pass2_call(E)(
        As, Dd, Bs, ef, p["edge_gate"]["W"], p["edge_gate"]["b"][None],
        p["bn_e"]["g"][None], p["bn_e"]["b"][None])
    SUMS = _sc_scatter_call(nseg, E, segsplit)(US, dst2, zrs)
    n_out = _pass3_call(nseg)(S, SUMS, p["bn_n"]["g"][None], p["bn_n"]["b"][None])
    return n_out, e_out


def kernel(atom_features, bondlength, angle_h, params, edge_index, lg_edge_index):
    n = atom_features.shape[0]
    m = bondlength.shape[0]
    t = angle_h.shape[0]
    src = edge_index[0].astype(jnp.int32)
    dst = edge_index[1].astype(jnp.int32)
    lsrc = lg_edge_index[0].astype(jnp.int32)
    ldst = lg_edge_index[1].astype(jnp.int32)
    src2 = src.reshape(m // BATCH, BATCH)
    dst2 = dst.reshape(m // BATCH, BATCH)
    lsrc2 = lsrc.reshape(t // BATCH, BATCH)
    ldst2 = ldst.reshape(t // BATCH, BATCH)
    zrs = jnp.zeros((ZROWS, 16), _f32)

    x = _atom_embed(atom_features, params["atom_emb"])
    y = _rbf_embed(bondlength, params["edge_m1"], params["edge_m2"], 0.0, 8.0, EBINS)
    z = _rbf_embed(angle_h, params["angle_m1"], params["angle_m2"], -1.0, 1.0, TBINS)

    for i in range(NAL):
        y, z = _egg(params["alignn"][i]["edge"], lsrc2, ldst2, y, z, m, t, zrs, True)
        x, y = _egg(params["alignn"][i]["node"], src2, dst2, x, y, n, m, zrs, False)
    for i in range(NGCN):
        x, y = _egg(params["gcn"][i], src2, dst2, x, y, n, m, zrs, False)
    return _mean_call(n)(x)


# speedup vs baseline: 1.0384x; 1.0384x over previous
import functools

import jax
import jax.numpy as jnp
from jax import lax
from jax.experimental import pallas as pl
from jax.experimental.pallas import tpu as pltpu
from jax.experimental.pallas import tpu_sc as plsc

HID = 128
EMB = 64
TBINS = 40
EBINS = 80
NAL = 4
NGCN = 4

NSC = 2
NTILE = 16
NW = NSC * NTILE

BATCH = 80
RPC = 8
CHUNK = BATCH * RPC
TBLK = 1000
ZROWS = 2500

_f32 = jnp.float32


def _ln(h, g, b):
    mu = jnp.mean(h, axis=-1, keepdims=True)
    var = jnp.mean((h - mu) ** 2, axis=-1, keepdims=True)
    return (h - mu) / jnp.sqrt(var + 1e-5) * g + b


def _rbf_body(v_ref, w1, b1, g1, e1, w2, b2, g2, e2, o_ref, *, vmin, vmax, bins):
    v = v_ref[...]
    step = (vmax - vmin) / (bins - 1)
    centers = vmin + lax.broadcasted_iota(jnp.int32, (1, bins), 1).astype(_f32) * step
    gamma = 0.5 / ((vmax - vmin) / bins) ** 2
    f = jnp.exp(-gamma * (v - centers) ** 2)
    h = jnp.dot(f, w1[...], preferred_element_type=_f32) + b1[...]
    h = jax.nn.silu(_ln(h, g1[...], e1[...]))
    h = jnp.dot(h, w2[...], preferred_element_type=_f32) + b2[...]
    o_ref[...] = jax.nn.silu(_ln(h, g2[...], e2[...]))


@functools.lru_cache(maxsize=None)
def _rbf_embed_call(n, bins, vmin, vmax):
    full = lambda shape: pl.BlockSpec(shape, lambda i: (0, 0))
    return pl.pallas_call(
        functools.partial(_rbf_body, vmin=vmin, vmax=vmax, bins=bins),
        grid=(n // TBLK,),
        in_specs=[
            pl.BlockSpec((TBLK, 1), lambda i: (i, 0)),
            full((bins, EMB)), full((1, EMB)), full((1, EMB)), full((1, EMB)),
            full((EMB, HID)), full((1, HID)), full((1, HID)), full((1, HID)),
        ],
        out_specs=pl.BlockSpec((TBLK, HID), lambda i: (i, 0)),
        out_shape=jax.ShapeDtypeStruct((n, HID), _f32),
    )


def _rbf_embed(vals, p1, p2, vmin, vmax, bins):
    n = vals.shape[0]
    return _rbf_embed_call(n, bins, vmin, vmax)(
        vals.reshape(n, 1),
        p1["lin"]["W"], p1["lin"]["b"][None], p1["ln"]["g"][None], p1["ln"]["b"][None],
        p2["lin"]["W"], p2["lin"]["b"][None], p2["ln"]["g"][None], p2["ln"]["b"][None],
    )


def _atom_body(x_ref, w, b, g, e, o_ref):
    h = jnp.dot(x_ref[...], w[...], preferred_element_type=_f32) + b[...]
    o_ref[...] = jax.nn.silu(_ln(h, g[...], e[...]))


@functools.lru_cache(maxsize=None)
def _atom_embed_call(n, din):
    full = lambda shape: pl.BlockSpec(shape, lambda i: (0, 0))
    return pl.pallas_call(
        _atom_body,
        grid=(n // TBLK,),
        in_specs=[
            pl.BlockSpec((TBLK, din), lambda i: (i, 0)),
            full((din, HID)), full((1, HID)), full((1, HID)), full((1, HID)),
        ],
        out_specs=pl.BlockSpec((TBLK, HID), lambda i: (i, 0)),
        out_shape=jax.ShapeDtypeStruct((n, HID), _f32),
    )


def _atom_embed(x, p):
    n, din = x.shape
    return _atom_embed_call(n, din)(
        x, p["lin"]["W"], p["lin"]["b"][None], p["ln"]["g"][None], p["ln"]["b"][None])


def _pass1_body(nf_ref, wcat, bcat, oa, od, ob, os_):
    h = jnp.dot(nf_ref[...], wcat[...], preferred_element_type=_f32) + bcat[...]
    oa[...] = h[:, 0 * HID:1 * HID]
    od[...] = h[:, 1 * HID:2 * HID]
    ob[...] = h[:, 2 * HID:3 * HID]
    os_[...] = h[:, 3 * HID:4 * HID]


@functools.lru_cache(maxsize=None)
def _pass1_call(n):
    full = lambda shape: pl.BlockSpec(shape, lambda i: (0, 0))
    blk = pl.BlockSpec((TBLK, HID), lambda i: (i, 0))
    return pl.pallas_call(
        _pass1_body,
        grid=(n // TBLK,),
        in_specs=[blk, full((HID, 4 * HID)), full((1, 4 * HID))],
        out_specs=[blk, blk, blk, blk],
        out_shape=[jax.ShapeDtypeStruct((n, HID), _f32)] * 4,
    )


def _pass2_body(as_ref, dd_ref, bs_ref, ef_ref, weg, beg, ge, be, us_ref, eo_ref):
    ef = ef_ref[...]
    eg = jnp.dot(ef, weg[...], preferred_element_type=_f32) + beg[...]
    sig = jax.nn.sigmoid(as_ref[...] + dd_ref[...] + eg)
    u = bs_ref[...] * sig
    us_ref[...] = jnp.concatenate([u, sig], axis=1)
    eo_ref[...] = _ln(sig * ef, ge[...], be[...])


@functools.lru_cache(maxsize=None)
def _pass2_call(E):
    full = lambda shape: pl.BlockSpec(shape, lambda i: (0, 0))
    blk = pl.BlockSpec((TBLK, HID), lambda i: (i, 0))
    return pl.pallas_call(
        _pass2_body,
        grid=(E // TBLK,),
        in_specs=[blk, blk, blk, blk,
                  full((HID, HID)), full((1, HID)), full((1, HID)), full((1, HID))],
        out_specs=[pl.BlockSpec((TBLK, 2 * HID), lambda i: (i, 0)), blk],
        out_shape=[jax.ShapeDtypeStruct((E, 2 * HID), _f32),
                   jax.ShapeDtypeStruct((E, HID), _f32)],
    )


def _pass3_body(s_ref, sums_ref, gn, bn, o_ref):
    sums = sums_ref[...]
    h = sums[:, :HID] / (sums[:, HID:] + 1e-8)
    o_ref[...] = _ln(s_ref[...] + h, gn[...], bn[...])


@functools.lru_cache(maxsize=None)
def _pass3_call(n):
    full = lambda shape: pl.BlockSpec(shape, lambda i: (0, 0))
    return pl.pallas_call(
        _pass3_body,
        grid=(n // TBLK,),
        in_specs=[pl.BlockSpec((TBLK, HID), lambda i: (i, 0)),
                  pl.BlockSpec((TBLK, 2 * HID), lambda i: (i, 0)),
                  full((1, HID)), full((1, HID))],
        out_specs=pl.BlockSpec((TBLK, HID), lambda i: (i, 0)),
        out_shape=jax.ShapeDtypeStruct((n, HID), _f32),
    )


def _mean_body(x_ref, o_ref, *, nblocks, n):
    i = pl.program_id(0)

    @pl.when(i == 0)
    def _():
        o_ref[...] = jnp.zeros_like(o_ref)

    o_ref[...] += jnp.sum(x_ref[...], axis=0, keepdims=True)

    @pl.when(i == nblocks - 1)
    def _():
        o_ref[...] *= 1.0 / n


@functools.lru_cache(maxsize=None)
def _mean_call(n):
    return pl.pallas_call(
        functools.partial(_mean_body, nblocks=n // TBLK, n=n),
        grid=(n // TBLK,),
        in_specs=[pl.BlockSpec((TBLK, HID), lambda i: (i, 0))],
        out_specs=pl.BlockSpec((1, HID), lambda i: (0, 0)),
        out_shape=jax.ShapeDtypeStruct((1, HID), _f32),
    )


def _sc_gather_body(A, D, B, src2, dst2, oA, oD, oB, sidx, didx, stag, sem,
                    *, nchunks, iters):
    cid = lax.axis_index("c")
    tid = lax.axis_index("s")
    wid = tid * NSC + cid

    def chunk_body(j, carry):
        k = j * NW + wid

        @pl.when(k < nchunks)
        def _():
            base = k * CHUNK
            row0 = k * RPC
            pltpu.sync_copy(src2.at[pl.ds(row0, RPC), :], sidx)
            pltpu.sync_copy(dst2.at[pl.ds(row0, RPC), :], didx)
            for tbl, idx, out in ((A, sidx, oA), (D, didx, oD), (B, sidx, oB)):
                descs = [
                    pltpu.async_copy(tbl.at[idx.at[r]],
                                     stag.at[pl.ds(r * BATCH, BATCH), :], sem)
                    for r in range(RPC)
                ]
                for d in descs:
                    d.wait()
                pltpu.sync_copy(stag, out.at[pl.ds(base, CHUNK), :])

        return carry

    lax.fori_loop(0, iters, chunk_body, 0)


@functools.lru_cache(maxsize=None)
def _sc_gather_call(n, E):
    nchunks = E // CHUNK
    iters = -(-nchunks // NW)
    mesh = plsc.VectorSubcoreMesh(core_axis_name="c", subcore_axis_name="s")
    return pl.kernel(
        functools.partial(_sc_gather_body, nchunks=nchunks, iters=iters),
        out_type=[jax.ShapeDtypeStruct((E, HID), _f32)] * 3,
        mesh=mesh,
        scratch_types=[
            pltpu.VMEM((RPC, BATCH), jnp.int32),
            pltpu.VMEM((RPC, BATCH), jnp.int32),
            pltpu.VMEM((CHUNK, HID), _f32),
            pltpu.SemaphoreType.DMA,
        ],
    )


def _sc_scatter_body(us, dst2, zrs, out, idxr, idxt, val, acc,
                     seml0, seml1, sems0, sems1,
                     *, nchunks, iters, half, segsplit, stripe):
    cid = lax.axis_index("c")
    tid = lax.axis_index("s")
    lo = cid * half if segsplit else 0
    ngroups = 16 if segsplit else 8
    seml = (seml0, seml1)
    sems = (sems0, sems1)

    def _load_descs(g, b, col0):
        k = g * NTILE + tid
        return k < nchunks, (
            (dst2.at[pl.ds(k * RPC, RPC), :], idxr.at[pl.ds(b * RPC, RPC), :], seml[b]),
            (us.at[pl.ds(k * CHUNK, CHUNK), pl.ds(col0, 16)],
             val.at[pl.ds(b * CHUNK, CHUNK), :], seml[b]),
        )

    def start_loads(g, b, col0):
        ok, descs = _load_descs(g, b, col0)

        @pl.when(ok)
        def _():
            for s, d, sm in descs:
                pltpu.async_copy(s, d, sm)

    def drain_loads(g, b, col0):
        ok, descs = _load_descs(g, b, col0)

        @pl.when(ok)
        def _():
            for s, d, sm in descs:
                pltpu.make_async_copy(s, d, sm).wait()

    def idx_rows(b):
        if not segsplit:
            return idxr
        return idxt

    def route_and_issue(g, b):
        k = g * NTILE + tid

        @pl.when(k < nchunks)
        def _():
            if segsplit:
                for r in range(RPC):
                    def route(c, cc):
                        v = idxr[b * RPC + r, pl.ds(c * 16, 16)]
                        vl = v - lo
                        ok = (vl >= 0) & (vl < half)
                        idxt[b * RPC + r, pl.ds(c * 16, 16)] = jnp.where(ok, vl, half)
                        return cc
                    lax.fori_loop(0, BATCH // 16, route, 0)
            rows = idx_rows(b)
            for r in range(RPC):
                pltpu.async_copy(val.at[pl.ds(b * CHUNK + r * BATCH, BATCH), :],
                                 acc.at[rows.at[b * RPC + r]], sems[b], add=True)

    def drain_scats(g, b):
        k = g * NTILE + tid

        @pl.when(k < nchunks)
        def _():
            rows = idx_rows(b)
            for r in range(RPC):
                pltpu.make_async_copy(val.at[pl.ds(b * CHUNK + r * BATCH, BATCH), :],
                                      acc.at[rows.at[b * RPC + r]], sems[b]).wait()

    npairs = (iters + 1) // 2

    for gg in range(ngroups):
        if segsplit:
            col0 = gg * 16
        else:
            col0 = (gg * NSC + cid) * 16
        nz = -(-stripe // ZROWS)
        for tz in range(nz):
            zr = min(ZROWS, stripe - tz * ZROWS)
            pltpu.sync_copy(zrs.at[pl.ds(0, zr), :],
                            acc.at[pl.ds(tid * stripe + tz * ZROWS, zr), :])
        plsc.subcore_barrier()

        start_loads(0, 0, col0)

        def pair_body(jj, carry):
            for h in range(2):
                g = jj * 2 + h
                drain_loads(g, h, col0)
                route_and_issue(g, h)
                @pl.when(g >= 1)
                def _():
                    drain_scats(g - 1, 1 - h)
                start_loads(g + 1, 1 - h, col0)
            return carry

        lax.fori_loop(0, npairs, pair_body, 0)
        drain_scats(2 * npairs - 1, 1)
        plsc.subcore_barrier()
        pltpu.sync_copy(acc.at[pl.ds(tid * stripe, stripe), :],
                        out.at[pl.ds(lo + tid * stripe, stripe), pl.ds(col0, 16)])
        plsc.subcore_barrier()


@functools.lru_cache(maxsize=None)
def _sc_scatter_call(nseg, E, segsplit):
    nchunks = E // CHUNK
    iters = -(-nchunks // NTILE)
    half = nseg // NSC if segsplit else nseg
    accrows = half + 8 if segsplit else nseg
    stripe = half // NTILE
    mesh = plsc.VectorSubcoreMesh(core_axis_name="c", subcore_axis_name="s")
    return pl.kernel(
        functools.partial(_sc_scatter_body, nchunks=nchunks, iters=iters,
                          half=half, segsplit=segsplit, stripe=stripe),
        out_type=jax.ShapeDtypeStruct((nseg, 2 * HID), _f32),
        mesh=mesh,
        scratch_types=[
            pltpu.VMEM((2 * RPC, BATCH), jnp.int32),
            pltpu.VMEM((2 * RPC, BATCH), jnp.int32),
            pltpu.VMEM((2 * CHUNK, 16), _f32),
            pltpu.VMEM_SHARED((accrows, 16), _f32),
            pltpu.SemaphoreType.DMA,
            pltpu.SemaphoreType.DMA,
            pltpu.SemaphoreType.DMA,
            pltpu.SemaphoreType.DMA,
        ],
        compiler_params=pltpu.CompilerParams(use_tc_tiling_on_sc=False),
    )


def _egg(p, src2, dst2, nf, ef, nseg, E, zrs, segsplit):
    wcat = jnp.concatenate([p["src_gate"]["W"], p["dst_gate"]["W"],
                            p["dst_update"]["W"], p["src_update"]["W"]], axis=1)
    bcat = jnp.concatenate([p["src_gate"]["b"], p["dst_gate"]["b"],
                            p["dst_update"]["b"], p["src_update"]["b"]])[None]
    A, D, B, S = _pass1_call(nseg)(nf, wcat, bcat)
    As, Dd, Bs = _sc_gather_call(nseg, E)(A, D, B, src2, dst2)
    US, e_out = _pass2_call(E)(
        As, Dd, Bs, ef, p["edge_gate"]["W"], p["edge_gate"]["b"][None],
        p["bn_e"]["g"][None], p["bn_e"]["b"][None])
    SUMS = _sc_scatter_call(nseg, E, segsplit)(US, dst2, zrs)
    n_out = _pass3_call(nseg)(S, SUMS, p["bn_n"]["g"][None], p["bn_n"]["b"][None])
    return n_out, e_out


def kernel(atom_features, bondlength, angle_h, params, edge_index, lg_edge_index):
    n = atom_features.shape[0]
    m = bondlength.shape[0]
    t = angle_h.shape[0]
    src = edge_index[0].astype(jnp.int32)
    dst = edge_index[1].astype(jnp.int32)
    lsrc = lg_edge_index[0].astype(jnp.int32)
    ldst = lg_edge_index[1].astype(jnp.int32)
    src2 = src.reshape(m // BATCH, BATCH)
    dst2 = dst.reshape(m // BATCH, BATCH)
    lsrc2 = lsrc.reshape(t // BATCH, BATCH)
    ldst2 = ldst.reshape(t // BATCH, BATCH)
    zrs = jnp.zeros((ZROWS, 16), _f32)

    x = _atom_embed(atom_features, params["atom_emb"])
    y = _rbf_embed(bondlength, params["edge_m1"], params["edge_m2"], 0.0, 8.0, EBINS)
    z = _rbf_embed(angle_h, params["angle_m1"], params["angle_m2"], -1.0, 1.0, TBINS)

    for i in range(NAL):
        y, z = _egg(params["alignn"][i]["edge"], lsrc2, ldst2, y, z, m, t, zrs, True)
        x, y = _egg(params["alignn"][i]["node"], src2, dst2, x, y, n, m, zrs, False)
    for i in range(NGCN):
        x, y = _egg(params["gcn"][i], src2, dst2, x, y, n, m, zrs, False)
    return _mean_call(n)(x)
